# baseline (speedup 1.0000x reference)
import jax
import jax.numpy as jnp
from jax import lax
from jax.experimental import pallas as pl
from jax.experimental.pallas import tpu as pltpu
from jax.experimental.pallas import tpu_sc as plsc

_INFO = plsc.get_sparse_core_info()
_NC = _INFO.num_cores
_NS = _INFO.num_subcores
_NW = _NC * _NS

_TABLE = 256 * 256 * 256
_B, _C, _H, _W = 4, 3, 512, 512
_PLANE = _H * _W
_PIX = _B * _PLANE
_PPW = _PIX // _NW
_CH = 2048
_NCHUNK = _PPW // _CH
_G = 128
_NG = _CH // _G
_NVEC = _CH // 16


def _sc_body(img_hbm, wt_hbm, bt_hbm, out_hbm,
             xbufs, idxbuf, wbufs, bbufs, obufs, sem):
    w_cols = tuple(wt_hbm.at[pl.ds(c * _TABLE, _TABLE)] for c in range(3))
    b_cols = tuple(bt_hbm.at[pl.ds(c * _TABLE, _TABLE)] for c in range(3))
    wid = lax.axis_index("s") * _NC + lax.axis_index("c")
    bi = wid // 8
    po = (wid % 8) * _PPW
    base = bi * (_C * _PLANE) + po

    def chunk_body(s, _):
        off = pl.multiple_of(base + s * _CH, 2048)
        for c in range(3):
            pltpu.sync_copy(img_hbm.at[pl.ds(off + c * _PLANE, _CH)],
                            xbufs[c])

        def idx_body(j, _):
            p = pl.ds(j * 16, 16)
            rv = xbufs[0][p]
            gv = xbufs[1][p]
            bv = xbufs[2][p]
            fidx = rv * 65536.0 + gv * 256.0 + bv
            idxbuf[p] = fidx.astype(jnp.int32)
            return 0

        lax.fori_loop(0, _NVEC, idx_body, 0, unroll=4)

        copies = []
        for g in range(_NG):
            gs = pl.ds(g * _G, _G)
            isl = idxbuf.at[gs]
            for c in range(3):
                copies.append(pltpu.async_copy(
                    w_cols[c].at[isl], wbufs[c].at[gs], sem))
                copies.append(pltpu.async_copy(
                    b_cols[c].at[isl], bbufs[c].at[gs], sem))
        for cp in copies:
            cp.wait()

        def fx_body(j, _):
            p = pl.ds(j * 16, 16)
            for c in range(3):
                wv = wbufs[c][p]
                bv = bbufs[c][p]
                xv = xbufs[c][p]
                obufs[c][p] = wv * xv + (bv - wv + 1.0) * 127.0
            return 0

        lax.fori_loop(0, _NVEC, fx_body, 0, unroll=4)

        for c in range(3):
            pltpu.sync_copy(obufs[c],
                            out_hbm.at[pl.ds(off + c * _PLANE, _CH)])
        return 0

    lax.fori_loop(0, _NCHUNK, chunk_body, 0)


@jax.jit
def kernel(img, w, b):
    img_flat = img.reshape(-1)
    mesh = plsc.VectorSubcoreMesh(core_axis_name="c", subcore_axis_name="s")
    out_flat = pl.kernel(
        _sc_body,
        out_type=jax.ShapeDtypeStruct((_B * _C * _PLANE,), jnp.float32),
        mesh=mesh,
        scratch_types=[
            [pltpu.VMEM((_CH,), jnp.float32)] * 3,
            pltpu.VMEM((_CH,), jnp.int32),
            [pltpu.VMEM((_CH,), jnp.float32)] * 3,
            [pltpu.VMEM((_CH,), jnp.float32)] * 3,
            [pltpu.VMEM((_CH,), jnp.float32)] * 3,
            pltpu.SemaphoreType.DMA,
        ],
    )(img_flat, w.T.reshape(-1), b.T.reshape(-1))
    return out_flat.reshape(_B, _C, _H, _W)


# speedup vs baseline: 5.3055x; 5.3055x over previous
import jax
import jax.numpy as jnp
from jax import lax
from jax.experimental import pallas as pl
from jax.experimental.pallas import tpu as pltpu
from jax.experimental.pallas import tpu_sc as plsc

_INFO = plsc.get_sparse_core_info()
_NC = _INFO.num_cores
_NS = _INFO.num_subcores
_NW = _NC * _NS

_B, _C, _H, _W = 4, 3, 512, 512
_PLANE = _H * _W
_PIX = _B * _PLANE
_PPW = _PIX // _NW
_CH = 2048
_NCHUNK = _PPW // _CH
_G = 512
_NG = _CH // _G
_NVEC = _CH // 16


def _sc_body(img_hbm, w0_hbm, w1_hbm, w2_hbm, b0_hbm, b1_hbm, b2_hbm, out_hbm,
             xbufs2, idxbuf2, wbufs2, bbufs2, obufs, gsem, isem):
    w_cols = (w0_hbm, w1_hbm, w2_hbm)
    b_cols = (b0_hbm, b1_hbm, b2_hbm)
    wid = lax.axis_index("s") * _NC + lax.axis_index("c")
    bi = wid // 8
    po = (wid % 8) * _PPW
    base = bi * (_C * _PLANE) + po

    def stage_in(s, p):
        off = base + s * _CH
        return [pltpu.async_copy(img_hbm.at[pl.ds(off + c * _PLANE, _CH)],
                                 xbufs2[p][c], isem)
                for c in range(3)]

    def do_idx(p):
        xb, ib = xbufs2[p], idxbuf2[p]

        def idx_body(j, _):
            q = pl.ds(j * 16, 16)
            fidx = xb[0][q] * 65536.0 + xb[1][q] * 256.0 + xb[2][q]
            ib[q] = fidx.astype(jnp.int32)
            return 0

        lax.fori_loop(0, _NVEC, idx_body, 0, unroll=4)

    def fire(p):
        ib, wb, bb = idxbuf2[p], wbufs2[p], bbufs2[p]
        hs = []
        for g in range(_NG):
            gs = pl.ds(g * _G, _G)
            isl = ib.at[gs]
            for c in range(3):
                hs.append(pltpu.async_copy(w_cols[c].at[isl],
                                           wb[c].at[gs], gsem))
                hs.append(pltpu.async_copy(b_cols[c].at[isl],
                                           bb[c].at[gs], gsem))
        return hs

    def finish(s, p):
        xb, wb, bb = xbufs2[p], wbufs2[p], bbufs2[p]

        def fx_body(j, _):
            q = pl.ds(j * 16, 16)
            for c in range(3):
                wv = wb[c][q]
                bv = bb[c][q]
                obufs[c][q] = wv * xb[c][q] + (bv - wv + 1.0) * 127.0
            return 0

        lax.fori_loop(0, _NVEC, fx_body, 0, unroll=4)
        off = base + s * _CH
        for c in range(3):
            pltpu.sync_copy(obufs[c],
                            out_hbm.at[pl.ds(off + c * _PLANE, _CH)])

    h_img = {0: stage_in(0, 0), 1: None}
    h_gat = {0: None, 1: None}
    prev = None
    for s in range(_NCHUNK):
        cur = s % 2
        for h in h_img[cur]:
            h.wait()
        do_idx(cur)
        h_gat[cur] = fire(cur)
        if prev is not None:
            pv = prev % 2
            for h in h_gat[pv]:
                h.wait()
            finish(prev, pv)
        if s + 1 < _NCHUNK:
            h_img[(s + 1) % 2] = stage_in(s + 1, (s + 1) % 2)
        prev = s
    pv = prev % 2
    for h in h_gat[pv]:
        h.wait()
    finish(prev, pv)


@jax.jit
def kernel(img, w, b):
    img_flat = img.reshape(-1)
    mesh = plsc.VectorSubcoreMesh(core_axis_name="c", subcore_axis_name="s")
    fbuf = pltpu.VMEM((_CH,), jnp.float32)
    out_flat = pl.kernel(
        _sc_body,
        out_type=jax.ShapeDtypeStruct((_B * _C * _PLANE,), jnp.float32),
        mesh=mesh,
        scratch_types=[
            [[fbuf] * 3] * 2,
            [pltpu.VMEM((_CH,), jnp.int32)] * 2,
            [[fbuf] * 3] * 2,
            [[fbuf] * 3] * 2,
            [fbuf] * 3,
            pltpu.SemaphoreType.DMA,
            pltpu.SemaphoreType.DMA,
        ],
    )(img_flat, w[:, 0], w[:, 1], w[:, 2], b[:, 0], b[:, 1], b[:, 2])
    return out_flat.reshape(_B, _C, _H, _W)


# speedup vs baseline: 5.3137x; 1.0015x over previous
import jax
import jax.numpy as jnp
from jax import lax
from jax.experimental import pallas as pl
from jax.experimental.pallas import tpu as pltpu
from jax.experimental.pallas import tpu_sc as plsc

_INFO = plsc.get_sparse_core_info()
_NC = _INFO.num_cores
_NS = _INFO.num_subcores
_NW = _NC * _NS

_B, _C, _H, _W = 4, 3, 512, 512
_PLANE = _H * _W
_PIX = _B * _PLANE
_PPW = _PIX // _NW
_CH = 2048
_NCHUNK = _PPW // _CH
_G = 2048
_NG = _CH // _G
_NVEC = _CH // 16


def _sc_body(img_hbm, w0_hbm, w1_hbm, w2_hbm, b0_hbm, b1_hbm, b2_hbm, out_hbm,
             xbufs2, idxbuf2, wbufs2, bbufs2, obufs, gsem, isem):
    w_cols = (w0_hbm, w1_hbm, w2_hbm)
    b_cols = (b0_hbm, b1_hbm, b2_hbm)
    wid = lax.axis_index("s") * _NC + lax.axis_index("c")
    bi = wid // 8
    po = (wid % 8) * _PPW
    base = bi * (_C * _PLANE) + po

    def stage_in(s, p):
        off = base + s * _CH
        return [pltpu.async_copy(img_hbm.at[pl.ds(off + c * _PLANE, _CH)],
                                 xbufs2[p][c], isem)
                for c in range(3)]

    def do_idx(p):
        xb, ib = xbufs2[p], idxbuf2[p]

        def idx_body(j, _):
            q = pl.ds(j * 16, 16)
            fidx = xb[0][q] * 65536.0 + xb[1][q] * 256.0 + xb[2][q]
            ib[q] = fidx.astype(jnp.int32)
            return 0

        lax.fori_loop(0, _NVEC, idx_body, 0, unroll=4)

    def fire(p):
        ib, wb, bb = idxbuf2[p], wbufs2[p], bbufs2[p]
        hs = []
        for g in range(_NG):
            gs = pl.ds(g * _G, _G)
            isl = ib.at[gs]
            for c in range(3):
                hs.append(pltpu.async_copy(w_cols[c].at[isl],
                                           wb[c].at[gs], gsem))
                hs.append(pltpu.async_copy(b_cols[c].at[isl],
                                           bb[c].at[gs], gsem))
        return hs

    def finish(s, p):
        xb, wb, bb = xbufs2[p], wbufs2[p], bbufs2[p]

        def fx_body(j, _):
            q = pl.ds(j * 16, 16)
            for c in range(3):
                wv = wb[c][q]
                bv = bb[c][q]
                obufs[c][q] = wv * xb[c][q] + (bv - wv + 1.0) * 127.0
            return 0

        lax.fori_loop(0, _NVEC, fx_body, 0, unroll=4)
        off = base + s * _CH
        for c in range(3):
            pltpu.sync_copy(obufs[c],
                            out_hbm.at[pl.ds(off + c * _PLANE, _CH)])

    h_img = {0: stage_in(0, 0), 1: None}
    h_gat = {0: None, 1: None}
    prev = None
    for s in range(_NCHUNK):
        cur = s % 2
        for h in h_img[cur]:
            h.wait()
        do_idx(cur)
        h_gat[cur] = fire(cur)
        if prev is not None:
            pv = prev % 2
            for h in h_gat[pv]:
                h.wait()
            finish(prev, pv)
        if s + 1 < _NCHUNK:
            h_img[(s + 1) % 2] = stage_in(s + 1, (s + 1) % 2)
        prev = s
    pv = prev % 2
    for h in h_gat[pv]:
        h.wait()
    finish(prev, pv)


@jax.jit
def kernel(img, w, b):
    img_flat = img.reshape(-1)
    mesh = plsc.VectorSubcoreMesh(core_axis_name="c", subcore_axis_name="s")
    fbuf = pltpu.VMEM((_CH,), jnp.float32)
    out_flat = pl.kernel(
        _sc_body,
        out_type=jax.ShapeDtypeStruct((_B * _C * _PLANE,), jnp.float32),
        mesh=mesh,
        scratch_types=[
            [[fbuf] * 3] * 2,
            [pltpu.VMEM((_CH,), jnp.int32)] * 2,
            [[fbuf] * 3] * 2,
            [[fbuf] * 3] * 2,
            [fbuf] * 3,
            pltpu.SemaphoreType.DMA,
            pltpu.SemaphoreType.DMA,
        ],
    )(img_flat, w[:, 0], w[:, 1], w[:, 2], b[:, 0], b[:, 1], b[:, 2])
    return out_flat.reshape(_B, _C, _H, _W)


# speedup vs baseline: 5.6174x; 1.0572x over previous
import jax
import jax.numpy as jnp
from jax import lax
from jax.experimental import pallas as pl
from jax.experimental.pallas import tpu as pltpu
from jax.experimental.pallas import tpu_sc as plsc

_INFO = plsc.get_sparse_core_info()
_NC = _INFO.num_cores
_NS = _INFO.num_subcores
_NW = _NC * _NS

_B, _C, _H, _W = 4, 3, 512, 512
_PLANE = _H * _W
_PIX = _B * _PLANE
_PPW = _PIX // _NW
_CH = 2048
_NCHUNK = _PPW // _CH
_G = 2048
_NG = _CH // _G
_NVEC = _CH // 16


def _worker_base():
    wid = lax.axis_index("s") * _NC + lax.axis_index("c")
    bi = wid // 8
    po = (wid % 8) * _PPW
    return bi * (_C * _PLANE) + po, bi * _PLANE + po


def _k1_body(img_hbm, b0_hbm, b1_hbm, b2_hbm, p1_hbm, idx_hbm,
             xbufs2, idxbuf2, bbufs2, obufs, gsem, isem):
    b_cols = (b0_hbm, b1_hbm, b2_hbm)
    base, pbase = _worker_base()

    def stage_in(s, p):
        off = base + s * _CH
        return [pltpu.async_copy(img_hbm.at[pl.ds(off + c * _PLANE, _CH)],
                                 xbufs2[p][c], isem)
                for c in range(3)]

    def do_idx(p):
        xb, ib = xbufs2[p], idxbuf2[p]

        def idx_body(j, _):
            q = pl.ds(j * 16, 16)
            fidx = xb[0][q] * 65536.0 + xb[1][q] * 256.0 + xb[2][q]
            ib[q] = fidx.astype(jnp.int32)
            return 0

        lax.fori_loop(0, _NVEC, idx_body, 0, unroll=4)

    def fire(p):
        ib, bb = idxbuf2[p], bbufs2[p]
        hs = []
        for g in range(_NG):
            gs = pl.ds(g * _G, _G)
            isl = ib.at[gs]
            for c in range(3):
                hs.append(pltpu.async_copy(b_cols[c].at[isl],
                                           bb[c].at[gs], gsem))
        return hs

    def finish(s, p):
        bb = bbufs2[p]

        def fx_body(j, _):
            q = pl.ds(j * 16, 16)
            for c in range(3):
                obufs[c][q] = (bb[c][q] + 1.0) * 127.0
            return 0

        lax.fori_loop(0, _NVEC, fx_body, 0, unroll=4)
        off = base + s * _CH
        for c in range(3):
            pltpu.sync_copy(obufs[c],
                            p1_hbm.at[pl.ds(off + c * _PLANE, _CH)])
        pltpu.sync_copy(idxbuf2[p],
                        idx_hbm.at[pl.ds(pbase + s * _CH, _CH)])

    h_img = {0: stage_in(0, 0), 1: None}
    h_gat = {0: None, 1: None}
    prev = None
    for s in range(_NCHUNK):
        cur = s % 2
        for h in h_img[cur]:
            h.wait()
        do_idx(cur)
        h_gat[cur] = fire(cur)
        if prev is not None:
            pv = prev % 2
            for h in h_gat[pv]:
                h.wait()
            finish(prev, pv)
        if s + 1 < _NCHUNK:
            h_img[(s + 1) % 2] = stage_in(s + 1, (s + 1) % 2)
        prev = s
    pv = prev % 2
    for h in h_gat[pv]:
        h.wait()
    finish(prev, pv)


def _k2_body(img_hbm, idx_hbm, p1_hbm, w0_hbm, w1_hbm, w2_hbm, out_hbm,
             xbufs2, pbufs2, idxbuf2, wbufs2, obufs, gsem, isem):
    w_cols = (w0_hbm, w1_hbm, w2_hbm)
    base, pbase = _worker_base()

    def stage_in(s, p):
        off = base + s * _CH
        hs = [pltpu.async_copy(img_hbm.at[pl.ds(off + c * _PLANE, _CH)],
                               xbufs2[p][c], isem)
              for c in range(3)]
        hs += [pltpu.async_copy(p1_hbm.at[pl.ds(off + c * _PLANE, _CH)],
                                pbufs2[p][c], isem)
               for c in range(3)]
        hs.append(pltpu.async_copy(idx_hbm.at[pl.ds(pbase + s * _CH, _CH)],
                                   idxbuf2[p], isem))
        return hs

    def fire(p):
        ib, wb = idxbuf2[p], wbufs2[p]
        hs = []
        for g in range(_NG):
            gs = pl.ds(g * _G, _G)
            isl = ib.at[gs]
            for c in range(3):
                hs.append(pltpu.async_copy(w_cols[c].at[isl],
                                           wb[c].at[gs], gsem))
        return hs

    def finish(s, p):
        xb, pb, wb = xbufs2[p], pbufs2[p], wbufs2[p]

        def fx_body(j, _):
            q = pl.ds(j * 16, 16)
            for c in range(3):
                obufs[c][q] = wb[c][q] * (xb[c][q] - 127.0) + pb[c][q]
            return 0

        lax.fori_loop(0, _NVEC, fx_body, 0, unroll=4)
        off = base + s * _CH
        for c in range(3):
            pltpu.sync_copy(obufs[c],
                            out_hbm.at[pl.ds(off + c * _PLANE, _CH)])

    h_img = {0: stage_in(0, 0), 1: None}
    h_gat = {0: None, 1: None}
    prev = None
    for s in range(_NCHUNK):
        cur = s % 2
        for h in h_img[cur]:
            h.wait()
        h_gat[cur] = fire(cur)
        if prev is not None:
            pv = prev % 2
            for h in h_gat[pv]:
                h.wait()
            finish(prev, pv)
        if s + 1 < _NCHUNK:
            h_img[(s + 1) % 2] = stage_in(s + 1, (s + 1) % 2)
        prev = s
    pv = prev % 2
    for h in h_gat[pv]:
        h.wait()
    finish(prev, pv)


@jax.jit
def kernel(img, w, b):
    img_flat = img.reshape(-1)
    mesh = plsc.VectorSubcoreMesh(core_axis_name="c", subcore_axis_name="s")
    fbuf = pltpu.VMEM((_CH,), jnp.float32)
    ibuf = pltpu.VMEM((_CH,), jnp.int32)
    p1, idx = pl.kernel(
        _k1_body,
        out_type=(jax.ShapeDtypeStruct((_B * _C * _PLANE,), jnp.float32),
                  jax.ShapeDtypeStruct((_PIX,), jnp.int32)),
        mesh=mesh,
        scratch_types=[
            [[fbuf] * 3] * 2,
            [ibuf] * 2,
            [[fbuf] * 3] * 2,
            [fbuf] * 3,
            pltpu.SemaphoreType.DMA,
            pltpu.SemaphoreType.DMA,
        ],
    )(img_flat, b[:, 0], b[:, 1], b[:, 2])
    out_flat = pl.kernel(
        _k2_body,
        out_type=jax.ShapeDtypeStruct((_B * _C * _PLANE,), jnp.float32),
        mesh=mesh,
        scratch_types=[
            [[fbuf] * 3] * 2,
            [[fbuf] * 3] * 2,
            [ibuf] * 2,
            [[fbuf] * 3] * 2,
            [fbuf] * 3,
            pltpu.SemaphoreType.DMA,
            pltpu.SemaphoreType.DMA,
        ],
    )(img_flat, idx, p1, w[:, 0], w[:, 1], w[:, 2])
    return out_flat.reshape(_B, _C, _H, _W)
